# baseline (speedup 1.0000x reference)
import jax
import jax.numpy as jnp
from jax.experimental import pallas as pl
from jax.experimental.pallas import tpu as pltpu

_NCH = 7
_CP = 232


def _chunks(P):
    out = []
    r = 0
    while r < P:
        n = min(_CP, P - r)
        out.append((r, n))
        r += n
    return out


def _body(ids_ref, hid_ref, emb_ref, gate_ref, out_ref,
          rows_ref, inb, outb, isem, osem):
    B, T, P, H = hid_ref.shape
    NSEG = B * T
    chunks = _chunks(P)

    g = jnp.tanh(gate_ref[...])
    for seg in range(NSEG):
        b, t = divmod(seg, T)
        rows_ref[seg] = emb_ref[ids_ref[b], t] * g

    def start_in(seg):
        b, t = divmod(seg, T)
        pg = seg % 2
        for row0, nrows in chunks:
            pltpu.make_async_copy(
                hid_ref.at[b, t, pl.ds(row0, nrows)],
                inb.at[pg, pl.ds(row0, nrows)],
                isem.at[pg],
            ).start()

    def wait_in(seg):
        b, t = divmod(seg, T)
        pg = seg % 2
        for row0, nrows in chunks:
            pltpu.make_async_copy(
                hid_ref.at[b, t, pl.ds(row0, nrows)],
                inb.at[pg, pl.ds(row0, nrows)],
                isem.at[pg],
            ).wait()

    def start_out(seg):
        b, t = divmod(seg, T)
        pg = seg % 2
        for row0, nrows in chunks:
            pltpu.make_async_copy(
                outb.at[pg, pl.ds(row0, nrows)],
                out_ref.at[b, t, pl.ds(row0, nrows)],
                osem.at[pg],
            ).start()

    def wait_out(seg):
        b, t = divmod(seg, T)
        pg = seg % 2
        for row0, nrows in chunks:
            pltpu.make_async_copy(
                outb.at[pg, pl.ds(row0, nrows)],
                out_ref.at[b, t, pl.ds(row0, nrows)],
                osem.at[pg],
            ).wait()

    start_in(0)
    start_in(1)
    for seg in range(NSEG):
        pg = seg % 2
        wait_in(seg)
        if seg >= 2:
            wait_out(seg - 2)
        outb[pg] = inb[pg] + rows_ref[seg]
        start_out(seg)
        if seg + 2 < NSEG:
            start_in(seg + 2)
    wait_out(NSEG - 2)
    wait_out(NSEG - 1)


def kernel(hidden_state, aspect_ratio_ids, embedding_table, gate):
    B, T, P, H = hidden_state.shape
    emb = embedding_table.reshape(-1, T, 1, H)
    ids = aspect_ratio_ids.astype(jnp.int32)
    gate2d = gate.reshape(1, 1)

    grid_spec = pltpu.PrefetchScalarGridSpec(
        num_scalar_prefetch=1,
        grid=(1,),
        in_specs=[
            pl.BlockSpec(memory_space=pl.ANY),
            pl.BlockSpec((emb.shape[0], T, 1, H), lambda i, ids_ref: (0, 0, 0, 0)),
            pl.BlockSpec((1, 1), lambda i, ids_ref: (0, 0)),
        ],
        out_specs=pl.BlockSpec(memory_space=pl.ANY),
        scratch_shapes=[
            pltpu.VMEM((B * T, 1, H), jnp.float32),
            pltpu.VMEM((2, P, H), jnp.float32),
            pltpu.VMEM((2, P, H), jnp.float32),
            pltpu.SemaphoreType.DMA((2,)),
            pltpu.SemaphoreType.DMA((2,)),
        ],
    )
    return pl.pallas_call(
        _body,
        grid_spec=grid_spec,
        out_shape=jax.ShapeDtypeStruct((B, T, P, H), hidden_state.dtype),
    )(ids, hidden_state, emb, gate2d)
